# Initial kernel scaffold; baseline (speedup 1.0000x reference)
import jax
import jax.numpy as jnp
from jax.experimental import pallas as pl


def kernel(x, x_cord, y_cord, one_player):
    raise NotImplementedError("write your pallas kernel here")


# speedup vs baseline: 1.1495x; 1.1495x over previous
import functools

import jax
import jax.numpy as jnp
import numpy as np
from jax import lax
from jax.experimental import pallas as pl
from jax.experimental.pallas import tpu as pltpu
from jax.experimental.pallas import tpu_sc as plsc

_N = 15
_M = 16
_B = 32
_C = 64
_H = 48
_W = 48
_NW = 32
_ROWS = _C * _N
_ROWS_PAD = 1024
_DATA = _C * _N * _N
_OUT_ROW = 14640


def _consts():
    k = np.arange(_ROWS_PAD)
    c48 = np.where(k < _ROWS, (k // _N) * _H, 0).astype(np.int32)
    k15 = np.where(k < _ROWS, k % _N, 0).astype(np.int32)
    p = np.arange(_DATA)
    idxrow = (p // _N).astype(np.int32)
    ij16 = (((p % (_N * _N)) // _N) * 16 + p % _N).astype(np.int32)
    f = np.arange(240)
    mij = np.where(f < _N * _N, (f // _N) * 16 + f % _N, 0).astype(np.int32)
    return c48, k15, idxrow, ij16, mij


def _sc_body(table, xs_all, ys_all, c48, k15, idxrow, ij16, mij, out,
             xs_v, ys_v, c48_v, k15_v, idxrow_v, ij16_v, mij_v,
             hc_v, col_v, mask_v, idxbuf, rowbuf, outbuf, sem):
    wid = lax.axis_index("s") * 2 + lax.axis_index("c")

    pltpu.sync_copy(xs_all.at[pl.ds(wid * _M, _M)], xs_v)
    pltpu.sync_copy(ys_all.at[pl.ds(wid * _M, _M)], ys_v)
    pltpu.sync_copy(c48, c48_v)
    pltpu.sync_copy(k15, k15_v)
    pltpu.sync_copy(idxrow, idxrow_v)
    pltpu.sync_copy(ij16, ij16_v)
    pltpu.sync_copy(mij, mij_v)

    iv = lax.iota(jnp.int32, 16)
    base = wid * (_C * _H)

    def patch_body(t, carry):
        tv = jnp.full((16,), t, jnp.int32)
        xsb = plsc.load_gather(xs_v, [tv])
        ysb = plsc.load_gather(ys_v, [tv])
        o = ysb - 7

        hc_v[...] = jnp.clip(xsb + iv - 7, 0, _H - 1)
        col_v[...] = jnp.clip(o + iv, 0, _W - 1)
        colok = (o + iv >= 0) & (o + iv < _W) & (iv < _N)
        for i in range(_N):
            rowok = (xsb >= 7 - i) & (xsb < _H + 7 - i)
            mask_v[pl.ds(i * 16, 16)] = jnp.where(
                colok & rowok, jnp.float32(1.0), jnp.float32(0.0))

        def idx_body(v, c):
            sl = pl.ds(v * 16, 16)
            hh = plsc.load_gather(hc_v, [k15_v[sl]])
            idxbuf[sl] = base + c48_v[sl] + hh
            return c
        lax.fori_loop(0, _ROWS_PAD // 16, idx_body, 0)

        cps = [
            pltpu.async_copy(table.at[idxbuf.at[pl.ds(d * 128, 128)]],
                             rowbuf.at[pl.ds(d * 128, 128)], sem)
            for d in range(_ROWS_PAD // 128)
        ]
        for cp in cps:
            cp.wait()

        def ext_body(p, c):
            sl = pl.ds(p * 16, 16)
            ij = ij16_v[sl]
            col = plsc.load_gather(col_v, [ij & 15])
            val = plsc.load_gather(rowbuf, [idxrow_v[sl], col])
            mf = plsc.load_gather(mask_v, [ij])
            outbuf[sl] = val * mf
            return c
        lax.fori_loop(0, _DATA // 16, ext_body, 0)

        def msk_body(q, c):
            outbuf[pl.ds(_DATA + q * 16, 16)] = plsc.load_gather(
                mask_v, [mij_v[pl.ds(q * 16, 16)]])
            return c
        lax.fori_loop(0, 15, msk_body, 0)

        pltpu.sync_copy(outbuf, out.at[wid * _M + t])
        return carry

    lax.fori_loop(0, _M, patch_body, 0)


def kernel(x, x_cord, y_cord, one_player):
    if one_player is not None:
        start = _M * jnp.asarray(one_player, dtype=jnp.int32)
        x_cord = lax.dynamic_slice_in_dim(x_cord, start, _M, axis=1)
        y_cord = lax.dynamic_slice_in_dim(y_cord, start, _M, axis=1)
    xs_all = x_cord.reshape(-1).astype(jnp.int32)
    ys_all = y_cord.reshape(-1).astype(jnp.int32)
    table = x.reshape(_B * _C * _H, _W)

    c48, k15, idxrow, ij16, mij = _consts()

    mesh = plsc.VectorSubcoreMesh(core_axis_name="c", subcore_axis_name="s")
    sc = functools.partial(
        pl.kernel,
        mesh=mesh,
        compiler_params=pltpu.CompilerParams(
            needs_layout_passes=False, use_tc_tiling_on_sc=False),
        out_type=jax.ShapeDtypeStruct((_B * _M, _OUT_ROW), jnp.float32),
        scratch_types=[
            pltpu.VMEM((_M,), jnp.int32),
            pltpu.VMEM((_M,), jnp.int32),
            pltpu.VMEM((_ROWS_PAD,), jnp.int32),
            pltpu.VMEM((_ROWS_PAD,), jnp.int32),
            pltpu.VMEM((_DATA,), jnp.int32),
            pltpu.VMEM((_DATA,), jnp.int32),
            pltpu.VMEM((240,), jnp.int32),
            pltpu.VMEM((16,), jnp.int32),
            pltpu.VMEM((16,), jnp.int32),
            pltpu.VMEM((240,), jnp.float32),
            pltpu.VMEM((_ROWS_PAD,), jnp.int32),
            pltpu.VMEM((_ROWS_PAD, _W), jnp.float32),
            pltpu.VMEM((_OUT_ROW,), jnp.float32),
            pltpu.SemaphoreType.DMA,
        ],
    )(_sc_body)

    out = sc(table, xs_all, ys_all,
             jnp.asarray(c48), jnp.asarray(k15), jnp.asarray(idxrow),
             jnp.asarray(ij16), jnp.asarray(mij))
    return out[:, :_C * _N * _N + _N * _N].reshape(_B * _M, _C + 1, _N, _N)


# speedup vs baseline: 1.6232x; 1.4121x over previous
import functools

import jax
import jax.numpy as jnp
import numpy as np
from jax import lax
from jax.experimental import pallas as pl
from jax.experimental.pallas import tpu as pltpu
from jax.experimental.pallas import tpu_sc as plsc

_N = 15
_M = 16
_B = 32
_C = 64
_H = 48
_W = 48
_NW = 32
_ROWS = _C * _N
_ROWS_PAD = 1024
_DATA = _C * _N * _N
_OUT = 14625
_OUT_ROW = 14640


def _consts():
    k = np.arange(_ROWS_PAD)
    c48 = np.where(k < _ROWS, (k // _N) * _H, 0).astype(np.int32)
    k15 = np.where(k < _ROWS, k % _N, 0).astype(np.int32)
    p = np.arange(_DATA)
    idxrow = (p // _N).astype(np.int32)
    ij16 = (((p % (_N * _N)) // _N) * 16 + p % _N).astype(np.int32)
    f = np.arange(240)
    mij = np.where(f < _N * _N, (f // _N) * 16 + f % _N, 0).astype(np.int32)
    return c48, k15, idxrow, ij16, mij


def _sc_body(table, xs_all, ys_all, c48, k15, idxrow, ij16, mij, out,
             xs_v, ys_v, c48_v, k15_v, idxrow_v, ij16_v, mij_v,
             hc_v, col_v, mask_v, idxbuf, rowbuf, outbuf, sem):
    wid = lax.axis_index("s") * 2 + lax.axis_index("c")

    pltpu.sync_copy(xs_all.at[pl.ds(wid * _M, _M)], xs_v)
    pltpu.sync_copy(ys_all.at[pl.ds(wid * _M, _M)], ys_v)
    pltpu.sync_copy(c48, c48_v)
    pltpu.sync_copy(k15, k15_v)
    pltpu.sync_copy(idxrow, idxrow_v)
    pltpu.sync_copy(ij16, ij16_v)
    pltpu.sync_copy(mij, mij_v)

    iv = lax.iota(jnp.int32, 16)
    base = wid * (_C * _H)

    def patch_body(t, carry):
        tv = jnp.full((16,), t, jnp.int32)
        xsb = plsc.load_gather(xs_v, [tv])
        ysb = plsc.load_gather(ys_v, [tv])
        o = ysb - 7

        hc_v[...] = jnp.clip(xsb + iv - 7, 0, _H - 1)
        col_v[...] = jnp.clip(o + iv, 0, _W - 1)
        colok = (o + iv >= 0) & (o + iv < _W) & (iv < _N)
        for i in range(_N):
            rowok = (xsb >= 7 - i) & (xsb < _H + 7 - i)
            mask_v[pl.ds(i * 16, 16)] = jnp.where(
                colok & rowok, jnp.float32(1.0), jnp.float32(0.0))

        @plsc.parallel_loop(0, _ROWS_PAD // 16, unroll=4)
        def idx_body(v):
            sl = pl.ds(v * 16, 16)
            hh = plsc.load_gather(hc_v, [k15_v[sl]])
            idxbuf[sl] = base + c48_v[sl] + hh

        cps = [
            pltpu.async_copy(table.at[idxbuf.at[pl.ds(d * 128, 128)]],
                             rowbuf.at[pl.ds(d * 128, 128)], sem)
            for d in range(_ROWS_PAD // 128)
        ]
        for cp in cps:
            cp.wait()

        @plsc.parallel_loop(0, _DATA // 16, unroll=8)
        def ext_body(p):
            sl = pl.ds(p * 16, 16)
            ij = ij16_v[sl]
            col = plsc.load_gather(col_v, [ij & 15])
            val = plsc.load_gather(rowbuf, [idxrow_v[sl], col])
            mf = plsc.load_gather(mask_v, [ij])
            outbuf[sl] = val * mf

        @plsc.parallel_loop(0, 15, unroll=5)
        def msk_body(q):
            outbuf[pl.ds(_DATA + q * 16, 16)] = plsc.load_gather(
                mask_v, [mij_v[pl.ds(q * 16, 16)]])

        pltpu.sync_copy(outbuf.at[pl.ds(0, _OUT)], out.at[wid * _M + t])
        return carry

    lax.fori_loop(0, _M, patch_body, 0)


def kernel(x, x_cord, y_cord, one_player):
    if one_player is not None:
        start = _M * jnp.asarray(one_player, dtype=jnp.int32)
        x_cord = lax.dynamic_slice_in_dim(x_cord, start, _M, axis=1)
        y_cord = lax.dynamic_slice_in_dim(y_cord, start, _M, axis=1)
    xs_all = x_cord.reshape(-1).astype(jnp.int32)
    ys_all = y_cord.reshape(-1).astype(jnp.int32)
    table = x.reshape(_B * _C * _H, _W)

    c48, k15, idxrow, ij16, mij = _consts()

    mesh = plsc.VectorSubcoreMesh(core_axis_name="c", subcore_axis_name="s")
    sc = functools.partial(
        pl.kernel,
        mesh=mesh,
        compiler_params=pltpu.CompilerParams(
            needs_layout_passes=False, use_tc_tiling_on_sc=False),
        out_type=jax.ShapeDtypeStruct((_B * _M, _OUT), jnp.float32),
        scratch_types=[
            pltpu.VMEM((_M,), jnp.int32),
            pltpu.VMEM((_M,), jnp.int32),
            pltpu.VMEM((_ROWS_PAD,), jnp.int32),
            pltpu.VMEM((_ROWS_PAD,), jnp.int32),
            pltpu.VMEM((_DATA,), jnp.int32),
            pltpu.VMEM((_DATA,), jnp.int32),
            pltpu.VMEM((240,), jnp.int32),
            pltpu.VMEM((16,), jnp.int32),
            pltpu.VMEM((16,), jnp.int32),
            pltpu.VMEM((240,), jnp.float32),
            pltpu.VMEM((_ROWS_PAD,), jnp.int32),
            pltpu.VMEM((_ROWS_PAD, _W), jnp.float32),
            pltpu.VMEM((_OUT_ROW,), jnp.float32),
            pltpu.SemaphoreType.DMA,
        ],
    )(_sc_body)

    out = sc(table, xs_all, ys_all,
             jnp.asarray(c48), jnp.asarray(k15), jnp.asarray(idxrow),
             jnp.asarray(ij16), jnp.asarray(mij))
    return out.reshape(_B * _M, _C + 1, _N, _N)


# speedup vs baseline: 2.0384x; 1.2558x over previous
import functools

import jax
import jax.numpy as jnp
import numpy as np
from jax import lax
from jax.experimental import pallas as pl
from jax.experimental.pallas import tpu as pltpu
from jax.experimental.pallas import tpu_sc as plsc

_N = 15
_M = 16
_B = 32
_C = 64
_CH = 32
_H = 48
_W = 48
_HP = _CH * _N * _N
_OUT = 14625


def _consts():
    p = np.arange(_HP)
    cl = ((p // (_N * _N)) * _H * _W).astype(np.int32)
    ij = (((p % (_N * _N)) // _N) * 16 + p % _N).astype(np.int32)
    f = np.arange(240)
    mij = np.where(f < _N * _N, (f // _N) * 16 + f % _N, 0).astype(np.int32)
    return cl, ij, mij


def _sc_body(table, xs_all, ys_all, cl, ij, mij, out,
             xs_v, ys_v, cl_v, ij_v, mij_v, hc_v, col_v, hcol_v, mask_v,
             mout_v, plane, hb, sem):
    wid = lax.axis_index("s") * 2 + lax.axis_index("c")

    pltpu.sync_copy(xs_all.at[pl.ds(wid * _M, _M)], xs_v)
    pltpu.sync_copy(ys_all.at[pl.ds(wid * _M, _M)], ys_v)
    pltpu.sync_copy(cl, cl_v)
    pltpu.sync_copy(ij, ij_v)
    pltpu.sync_copy(mij, mij_v)

    iv = lax.iota(jnp.int32, 16)

    for half in range(2):
        base0 = (wid * _C + half * _CH) * _H * _W
        pltpu.sync_copy(table.at[pl.ds(base0, _CH * _H * _W)], plane)

        def patch_body(t, carry):
            tv = jnp.full((16,), t, jnp.int32)
            xsb = plsc.load_gather(xs_v, [tv])
            ysb = plsc.load_gather(ys_v, [tv])
            o = ysb - 7

            hc_v[...] = jnp.clip(xsb + iv - 7, 0, _H - 1) * _W
            col_v[...] = jnp.clip(o + iv, 0, _W - 1)
            colok = (o + iv >= 0) & (o + iv < _W) & (iv < _N)
            colokf = jnp.where(colok, jnp.float32(1.0), jnp.float32(0.0))
            colv = col_v[...]
            for i in range(_N):
                hrow = plsc.load_gather(hc_v, [jnp.full((16,), i, jnp.int32)])
                hcol_v[pl.ds(i * 16, 16)] = hrow + colv
                rowok = (xsb >= 7 - i) & (xsb < _H + 7 - i)
                rowokf = jnp.where(rowok, jnp.float32(1.0), jnp.float32(0.0))
                mask_v[pl.ds(i * 16, 16)] = colokf * rowokf

            @plsc.parallel_loop(0, _HP // 16, unroll=8)
            def ext_body(p):
                sl = pl.ds(p * 16, 16)
                idx = cl_v[sl] + plsc.load_gather(hcol_v, [ij_v[sl]])
                val = plsc.load_gather(plane, [idx])
                mf = plsc.load_gather(mask_v, [ij_v[sl]])
                hb[sl] = val * mf

            bm = wid * _M + t
            pltpu.sync_copy(hb.at[pl.ds(0, _HP)],
                            out.at[bm, pl.ds(half * _HP, _HP)])
            if half == 1:
                @plsc.parallel_loop(0, 15, unroll=5)
                def msk_body(q):
                    mout_v[pl.ds(q * 16, 16)] = plsc.load_gather(
                        mask_v, [mij_v[pl.ds(q * 16, 16)]])
                pltpu.sync_copy(mout_v.at[pl.ds(0, _N * _N)],
                                out.at[bm, pl.ds(2 * _HP, _N * _N)])
            return carry

        lax.fori_loop(0, _M, patch_body, 0)


def kernel(x, x_cord, y_cord, one_player):
    if one_player is not None:
        start = _M * jnp.asarray(one_player, dtype=jnp.int32)
        x_cord = lax.dynamic_slice_in_dim(x_cord, start, _M, axis=1)
        y_cord = lax.dynamic_slice_in_dim(y_cord, start, _M, axis=1)
    xs_all = x_cord.reshape(-1).astype(jnp.int32)
    ys_all = y_cord.reshape(-1).astype(jnp.int32)
    table = x.reshape(_B * _C * _H * _W)

    cl, ij, mij = _consts()

    mesh = plsc.VectorSubcoreMesh(core_axis_name="c", subcore_axis_name="s")
    sc = functools.partial(
        pl.kernel,
        mesh=mesh,
        compiler_params=pltpu.CompilerParams(
            needs_layout_passes=False, use_tc_tiling_on_sc=False),
        out_type=jax.ShapeDtypeStruct((_B * _M, _OUT), jnp.float32),
        scratch_types=[
            pltpu.VMEM((_M,), jnp.int32),
            pltpu.VMEM((_M,), jnp.int32),
            pltpu.VMEM((_HP,), jnp.int32),
            pltpu.VMEM((_HP,), jnp.int32),
            pltpu.VMEM((240,), jnp.int32),
            pltpu.VMEM((16,), jnp.int32),
            pltpu.VMEM((16,), jnp.int32),
            pltpu.VMEM((240,), jnp.int32),
            pltpu.VMEM((240,), jnp.float32),
            pltpu.VMEM((240,), jnp.float32),
            pltpu.VMEM((_CH * _H * _W,), jnp.float32),
            pltpu.VMEM((_HP,), jnp.float32),
            pltpu.SemaphoreType.DMA,
        ],
    )(_sc_body)

    out = sc(table, xs_all, ys_all,
             jnp.asarray(cl), jnp.asarray(ij), jnp.asarray(mij))
    return out.reshape(_B * _M, _C + 1, _N, _N)


# speedup vs baseline: 2.0462x; 1.0039x over previous
import functools

import jax
import jax.numpy as jnp
import numpy as np
from jax import lax
from jax.experimental import pallas as pl
from jax.experimental.pallas import tpu as pltpu
from jax.experimental.pallas import tpu_sc as plsc

_N = 15
_M = 16
_B = 32
_C = 64
_CH = 32
_H = 48
_W = 48
_HP = _CH * _N * _N
_OUT = 14625


def _consts():
    p = np.arange(_HP)
    cl = ((p // (_N * _N)) * _H * _W).astype(np.int32)
    ij = (((p % (_N * _N)) // _N) * 16 + p % _N).astype(np.int32)
    f = np.arange(240)
    mij = np.where(f < _N * _N, (f // _N) * 16 + f % _N, 0).astype(np.int32)
    return cl, ij, mij


def _sc_body(table, xs_all, ys_all, cl, ij, mij, out,
             xs_v, ys_v, cl_v, ij_v, mij_v, hcol_v, mask_v,
             mout_v, plane, hb, sem):
    wid = lax.axis_index("s") * 2 + lax.axis_index("c")

    pltpu.sync_copy(xs_all.at[pl.ds(wid * _M, _M)], xs_v)
    pltpu.sync_copy(ys_all.at[pl.ds(wid * _M, _M)], ys_v)
    pltpu.sync_copy(cl, cl_v)
    pltpu.sync_copy(ij, ij_v)
    pltpu.sync_copy(mij, mij_v)

    iv = lax.iota(jnp.int32, 16)

    for half in range(2):
        base0 = (wid * _C + half * _CH) * _H * _W
        pltpu.sync_copy(table.at[pl.ds(base0, _CH * _H * _W)], plane)

        def patch_body(t, carry):
            tv = jnp.full((16,), t, jnp.int32)
            xsb = plsc.load_gather(xs_v, [tv])
            ysb = plsc.load_gather(ys_v, [tv])
            o = ysb - 7

            colv = jnp.clip(o + iv, 0, _W - 1)
            colok = (o + iv >= 0) & (o + iv < _W) & (iv < _N)
            colokf = jnp.where(colok, jnp.float32(1.0), jnp.float32(0.0))
            for i in range(_N):
                hrow = jnp.clip(xsb + (i - 7), 0, _H - 1) * _W
                hcol_v[pl.ds(i * 16, 16)] = hrow + colv
                rowok = (xsb >= 7 - i) & (xsb < _H + 7 - i)
                rowokf = jnp.where(rowok, jnp.float32(1.0), jnp.float32(0.0))
                mask_v[pl.ds(i * 16, 16)] = colokf * rowokf

            @plsc.parallel_loop(0, _HP // 16, unroll=8)
            def ext_body(p):
                sl = pl.ds(p * 16, 16)
                idx = cl_v[sl] + plsc.load_gather(hcol_v, [ij_v[sl]])
                val = plsc.load_gather(plane, [idx])
                mf = plsc.load_gather(mask_v, [ij_v[sl]])
                hb[sl] = val * mf

            bm = wid * _M + t
            pltpu.sync_copy(hb.at[pl.ds(0, _HP)],
                            out.at[bm, pl.ds(half * _HP, _HP)])
            if half == 1:
                @plsc.parallel_loop(0, 15, unroll=5)
                def msk_body(q):
                    mout_v[pl.ds(q * 16, 16)] = plsc.load_gather(
                        mask_v, [mij_v[pl.ds(q * 16, 16)]])
                pltpu.sync_copy(mout_v.at[pl.ds(0, _N * _N)],
                                out.at[bm, pl.ds(2 * _HP, _N * _N)])
            return carry

        lax.fori_loop(0, _M, patch_body, 0)


def kernel(x, x_cord, y_cord, one_player):
    if one_player is not None:
        start = _M * jnp.asarray(one_player, dtype=jnp.int32)
        x_cord = lax.dynamic_slice_in_dim(x_cord, start, _M, axis=1)
        y_cord = lax.dynamic_slice_in_dim(y_cord, start, _M, axis=1)
    xs_all = x_cord.reshape(-1).astype(jnp.int32)
    ys_all = y_cord.reshape(-1).astype(jnp.int32)
    table = x.reshape(_B * _C * _H * _W)

    cl, ij, mij = _consts()

    mesh = plsc.VectorSubcoreMesh(core_axis_name="c", subcore_axis_name="s")
    sc = functools.partial(
        pl.kernel,
        mesh=mesh,
        compiler_params=pltpu.CompilerParams(
            needs_layout_passes=False, use_tc_tiling_on_sc=False),
        out_type=jax.ShapeDtypeStruct((_B * _M, _OUT), jnp.float32),
        scratch_types=[
            pltpu.VMEM((_M,), jnp.int32),
            pltpu.VMEM((_M,), jnp.int32),
            pltpu.VMEM((_HP,), jnp.int32),
            pltpu.VMEM((_HP,), jnp.int32),
            pltpu.VMEM((240,), jnp.int32),
            pltpu.VMEM((240,), jnp.int32),
            pltpu.VMEM((240,), jnp.float32),
            pltpu.VMEM((240,), jnp.float32),
            pltpu.VMEM((_CH * _H * _W,), jnp.float32),
            pltpu.VMEM((_HP,), jnp.float32),
            pltpu.SemaphoreType.DMA,
        ],
    )(_sc_body)

    out = sc(table, xs_all, ys_all,
             jnp.asarray(cl), jnp.asarray(ij), jnp.asarray(mij))
    return out.reshape(_B * _M, _C + 1, _N, _N)


# speedup vs baseline: 2.1657x; 1.0584x over previous
import functools

import jax
import jax.numpy as jnp
import numpy as np
from jax import lax
from jax.experimental import pallas as pl
from jax.experimental.pallas import tpu as pltpu
from jax.experimental.pallas import tpu_sc as plsc

_N = 15
_M = 16
_B = 32
_C = 64
_CH = 32
_H = 48
_W = 48
_HP = _CH * _N * _N
_OUT = 14625
_PL = _CH * _H * _W
_BIG = 1 << 20


def _consts():
    p = np.arange(_HP)
    cl = (p // (_N * _N)) * _H * _W
    ij = ((p % (_N * _N)) // _N) * 16 + p % _N
    pk = ((cl << 8) | ij).astype(np.int32)
    f = np.arange(240)
    mij = np.where(f < _N * _N, (f // _N) * 16 + f % _N, 0).astype(np.int32)
    return pk, mij


def _sc_body(table, xs_all, ys_all, pk, mij, out,
             xs_v, ys_v, pk_v, mij_v, hcol_v, mask_v,
             mout_v, plane, hb, sem):
    wid = lax.axis_index("s") * 2 + lax.axis_index("c")

    pltpu.sync_copy(xs_all.at[pl.ds(wid * _M, _M)], xs_v)
    pltpu.sync_copy(ys_all.at[pl.ds(wid * _M, _M)], ys_v)
    pltpu.sync_copy(pk, pk_v)
    pltpu.sync_copy(mij, mij_v)

    iv = lax.iota(jnp.int32, 16)
    plane[pl.ds(_PL, 16)] = jnp.zeros((16,), jnp.float32)

    for half in range(2):
        base0 = (wid * _C + half * _CH) * _H * _W
        pltpu.sync_copy(table.at[pl.ds(base0, _PL)], plane.at[pl.ds(0, _PL)])

        def patch_body(t, carry):
            tv = jnp.full((16,), t, jnp.int32)
            xsb = plsc.load_gather(xs_v, [tv])
            ysb = plsc.load_gather(ys_v, [tv])
            o = ysb - 7

            colv = jnp.clip(o + iv, 0, _W - 1)
            colok = (o + iv >= 0) & (o + iv < _W) & (iv < _N)
            for i in range(_N):
                hrow = jnp.clip(xsb + (i - 7), 0, _H - 1) * _W
                rowok = (xsb >= 7 - i) & (xsb < _H + 7 - i)
                hcol_v[pl.ds(i * 16, 16)] = jnp.where(
                    colok & rowok, hrow + colv, jnp.int32(_BIG))
                if half == 1:
                    mask_v[pl.ds(i * 16, 16)] = jnp.where(
                        colok & rowok, jnp.float32(1.0), jnp.float32(0.0))

            @plsc.parallel_loop(0, _HP // 16, unroll=8)
            def ext_body(p):
                sl = pl.ds(p * 16, 16)
                pkv = pk_v[sl]
                idx = (pkv >> 8) + plsc.load_gather(hcol_v, [pkv & 255])
                idx = jnp.minimum(idx, _PL)
                hb[sl] = plsc.load_gather(plane, [idx])

            bm = wid * _M + t
            pltpu.sync_copy(hb.at[pl.ds(0, _HP)],
                            out.at[bm, pl.ds(half * _HP, _HP)])
            if half == 1:
                @plsc.parallel_loop(0, 15, unroll=5)
                def msk_body(q):
                    mout_v[pl.ds(q * 16, 16)] = plsc.load_gather(
                        mask_v, [mij_v[pl.ds(q * 16, 16)]])
                pltpu.sync_copy(mout_v.at[pl.ds(0, _N * _N)],
                                out.at[bm, pl.ds(2 * _HP, _N * _N)])
            return carry

        lax.fori_loop(0, _M, patch_body, 0)


def kernel(x, x_cord, y_cord, one_player):
    if one_player is not None:
        start = _M * jnp.asarray(one_player, dtype=jnp.int32)
        x_cord = lax.dynamic_slice_in_dim(x_cord, start, _M, axis=1)
        y_cord = lax.dynamic_slice_in_dim(y_cord, start, _M, axis=1)
    xs_all = x_cord.reshape(-1).astype(jnp.int32)
    ys_all = y_cord.reshape(-1).astype(jnp.int32)
    table = x.reshape(_B * _C * _H * _W)

    pk, mij = _consts()

    mesh = plsc.VectorSubcoreMesh(core_axis_name="c", subcore_axis_name="s")
    sc = functools.partial(
        pl.kernel,
        mesh=mesh,
        compiler_params=pltpu.CompilerParams(
            needs_layout_passes=False, use_tc_tiling_on_sc=False),
        out_type=jax.ShapeDtypeStruct((_B * _M, _OUT), jnp.float32),
        scratch_types=[
            pltpu.VMEM((_M,), jnp.int32),
            pltpu.VMEM((_M,), jnp.int32),
            pltpu.VMEM((_HP,), jnp.int32),
            pltpu.VMEM((240,), jnp.int32),
            pltpu.VMEM((240,), jnp.int32),
            pltpu.VMEM((240,), jnp.float32),
            pltpu.VMEM((240,), jnp.float32),
            pltpu.VMEM((_PL + 16,), jnp.float32),
            pltpu.VMEM((_HP,), jnp.float32),
            pltpu.SemaphoreType.DMA,
        ],
    )(_sc_body)

    out = sc(table, xs_all, ys_all, jnp.asarray(pk), jnp.asarray(mij))
    return out.reshape(_B * _M, _C + 1, _N, _N)


# speedup vs baseline: 2.2551x; 1.0413x over previous
import functools

import jax
import jax.numpy as jnp
import numpy as np
from jax import lax
from jax.experimental import pallas as pl
from jax.experimental.pallas import tpu as pltpu
from jax.experimental.pallas import tpu_sc as plsc

_N = 15
_M = 16
_B = 32
_C = 64
_CH = 32
_H = 48
_W = 48
_HP = _CH * _N * _N
_HB = 7440
_OUT = 14625
_PL = _CH * _H * _W
_BIG = 1 << 20


def _consts():
    p = np.arange(_HP)
    cl = (p // (_N * _N)) * _H * _W
    ij = ((p % (_N * _N)) // _N) * 16 + p % _N
    pk = ((cl << 8) | ij).astype(np.int32)
    f = np.arange(240)
    mij = np.where(f < _N * _N, (f // _N) * 16 + f % _N, 0).astype(np.int32)
    return pk, mij


def _sc_body(table, xs_all, ys_all, pk, mij, out,
             xs_v, ys_v, pk_v, mij_v, hcol_v, mask_v, mout_v, plane, hb, sem):
    wid = lax.axis_index("s") * 2 + lax.axis_index("c")

    pltpu.sync_copy(xs_all.at[pl.ds(wid * _M, _M)], xs_v)
    pltpu.sync_copy(ys_all.at[pl.ds(wid * _M, _M)], ys_v)
    pltpu.sync_copy(pk, pk_v)
    pltpu.sync_copy(mij, mij_v)

    iv = lax.iota(jnp.int32, 16)
    plane[pl.ds(_PL, 16)] = jnp.zeros((16,), jnp.float32)

    for half in range(2):
        base0 = (wid * _C + half * _CH) * _H * _W
        pltpu.sync_copy(table.at[pl.ds(base0, _PL)], plane.at[pl.ds(0, _PL)])

        def patch_body(t, carry):
            tb = t & 1

            @pl.when(t >= 2)
            def _drain():
                pltpu.make_async_copy(
                    table.at[pl.ds(0, _HP)], hb.at[tb, pl.ds(0, _HP)],
                    sem).wait()

            tv = jnp.full((16,), t, jnp.int32)
            xsb = plsc.load_gather(xs_v, [tv])
            ysb = plsc.load_gather(ys_v, [tv])
            o = ysb - 7

            colv = jnp.clip(o + iv, 0, _W - 1)
            colok = (o + iv >= 0) & (o + iv < _W) & (iv < _N)
            for i in range(_N):
                hrow = jnp.clip(xsb + (i - 7), 0, _H - 1) * _W
                rowok = (xsb >= 7 - i) & (xsb < _H + 7 - i)
                hcol_v[pl.ds(i * 16, 16)] = jnp.where(
                    colok & rowok, hrow + colv, jnp.int32(_BIG))
                if half == 1:
                    mask_v[pl.ds(i * 16, 16)] = jnp.where(
                        colok & rowok, jnp.float32(1.0), jnp.float32(0.0))

            hbv = hb.at[tb]

            @plsc.parallel_loop(0, _HP // 16, unroll=8)
            def ext_body(p):
                sl = pl.ds(p * 16, 16)
                pkv = pk_v[sl]
                idx = (pkv >> 8) + plsc.load_gather(hcol_v, [pkv & 255])
                idx = jnp.minimum(idx, _PL)
                hbv[sl] = plsc.load_gather(plane, [idx])

            bm = wid * _M + t
            pltpu.async_copy(hb.at[tb, pl.ds(0, _HP)],
                             out.at[bm, pl.ds(half * _HP, _HP)], sem)
            if half == 1:
                @plsc.parallel_loop(0, 15, unroll=5)
                def msk_body(q):
                    mout_v[pl.ds(q * 16, 16)] = plsc.load_gather(
                        mask_v, [mij_v[pl.ds(q * 16, 16)]])
                pltpu.sync_copy(mout_v.at[pl.ds(0, _N * _N)],
                                out.at[bm, pl.ds(2 * _HP, _N * _N)])
            return carry

        lax.fori_loop(0, _M, patch_body, 0)
        for k in range(2):
            pltpu.make_async_copy(
                table.at[pl.ds(0, _HP)], hb.at[k, pl.ds(0, _HP)], sem).wait()


def kernel(x, x_cord, y_cord, one_player):
    if one_player is not None:
        start = _M * jnp.asarray(one_player, dtype=jnp.int32)
        x_cord = lax.dynamic_slice_in_dim(x_cord, start, _M, axis=1)
        y_cord = lax.dynamic_slice_in_dim(y_cord, start, _M, axis=1)
    xs_all = x_cord.reshape(-1).astype(jnp.int32)
    ys_all = y_cord.reshape(-1).astype(jnp.int32)
    table = x.reshape(_B * _C * _H * _W)

    pk, mij = _consts()

    mesh = plsc.VectorSubcoreMesh(core_axis_name="c", subcore_axis_name="s")
    sc = functools.partial(
        pl.kernel,
        mesh=mesh,
        compiler_params=pltpu.CompilerParams(
            needs_layout_passes=False, use_tc_tiling_on_sc=False,
            skip_device_barrier=True),
        out_type=jax.ShapeDtypeStruct((_B * _M, _OUT), jnp.float32),
        scratch_types=[
            pltpu.VMEM((_M,), jnp.int32),
            pltpu.VMEM((_M,), jnp.int32),
            pltpu.VMEM((_HP,), jnp.int32),
            pltpu.VMEM((240,), jnp.int32),
            pltpu.VMEM((240,), jnp.int32),
            pltpu.VMEM((240,), jnp.float32),
            pltpu.VMEM((240,), jnp.float32),
            pltpu.VMEM((_PL + 16,), jnp.float32),
            pltpu.VMEM((2, _HB), jnp.float32),
            pltpu.SemaphoreType.DMA,
        ],
    )(_sc_body)

    out = sc(table, xs_all, ys_all, jnp.asarray(pk), jnp.asarray(mij))
    return out.reshape(_B * _M, _C + 1, _N, _N)


# speedup vs baseline: 2.2596x; 1.0020x over previous
import functools

import jax
import jax.numpy as jnp
import numpy as np
from jax import lax
from jax.experimental import pallas as pl
from jax.experimental.pallas import tpu as pltpu
from jax.experimental.pallas import tpu_sc as plsc

_N = 15
_M = 16
_B = 32
_C = 64
_CH = 32
_H = 48
_W = 48
_HP = _CH * _N * _N
_HB = 7440
_OUT = 14625
_OUTP = 14632
_PL = _CH * _H * _W
_BIG = 1 << 20


def _consts():
    p = np.arange(_HP)
    cl = (p // (_N * _N)) * _H * _W
    ij = ((p % (_N * _N)) // _N) * 16 + p % _N
    pk = ((cl << 8) | ij).astype(np.int32)
    f = np.arange(240)
    mij = np.where(f < _N * _N, (f // _N) * 16 + f % _N, 0).astype(np.int32)
    return pk, mij


def _sc_body(table, xs_all, ys_all, pk, mij, out,
             xs_v, ys_v, pk_v, mij_v, hcol_v, mask_v, mout_v, plane, hb, sem):
    wid = lax.axis_index("s") * 2 + lax.axis_index("c")

    pltpu.sync_copy(xs_all.at[pl.ds(wid * _M, _M)], xs_v)
    pltpu.sync_copy(ys_all.at[pl.ds(wid * _M, _M)], ys_v)
    pltpu.sync_copy(pk, pk_v)
    pltpu.sync_copy(mij, mij_v)

    iv = lax.iota(jnp.int32, 16)
    plane[pl.ds(_PL, 16)] = jnp.zeros((16,), jnp.float32)

    def half_body(half, hcarry):
        base0 = (wid * _C + half * _CH) * _H * _W
        pltpu.sync_copy(table.at[pl.ds(base0, _PL)], plane.at[pl.ds(0, _PL)])

        def patch_body(t, carry):
            tb = t & 1

            @pl.when(t >= 2)
            def _drain():
                pltpu.make_async_copy(
                    table.at[pl.ds(0, _HP)], hb.at[tb, pl.ds(0, _HP)],
                    sem).wait()

            tv = jnp.full((16,), t, jnp.int32)
            xsb = plsc.load_gather(xs_v, [tv])
            ysb = plsc.load_gather(ys_v, [tv])
            o = ysb - 7

            colv = jnp.clip(o + iv, 0, _W - 1)
            colok = (o + iv >= 0) & (o + iv < _W) & (iv < _N)
            for i in range(_N):
                hrow = jnp.clip(xsb + (i - 7), 0, _H - 1) * _W
                rowok = (xsb >= 7 - i) & (xsb < _H + 7 - i)
                hcol_v[pl.ds(i * 16, 16)] = jnp.where(
                    colok & rowok, hrow + colv, jnp.int32(_BIG))
                mask_v[pl.ds(i * 16, 16)] = jnp.where(
                    colok & rowok, jnp.float32(1.0), jnp.float32(0.0))

            hbv = hb.at[tb]

            @plsc.parallel_loop(0, _HP // 16, unroll=8)
            def ext_body(p):
                sl = pl.ds(p * 16, 16)
                pkv = pk_v[sl]
                idx = (pkv >> 8) + plsc.load_gather(hcol_v, [pkv & 255])
                idx = jnp.minimum(idx, _PL)
                hbv[sl] = plsc.load_gather(plane, [idx])

            bm = wid * _M + t
            obase = bm * _OUTP + half * _HP
            pltpu.async_copy(hb.at[tb, pl.ds(0, _HP)],
                             out.at[pl.ds(obase, _HP)], sem)

            @pl.when(half == 1)
            def _mask_out():
                @plsc.parallel_loop(0, 15, unroll=5)
                def msk_body(q):
                    mout_v[pl.ds(q * 16, 16)] = plsc.load_gather(
                        mask_v, [mij_v[pl.ds(q * 16, 16)]])
                pltpu.sync_copy(mout_v.at[pl.ds(0, _N * _N)],
                                out.at[pl.ds(bm * _OUTP + 2 * _HP, _N * _N)])
            return carry

        lax.fori_loop(0, _M, patch_body, 0)
        for k in range(2):
            pltpu.make_async_copy(
                table.at[pl.ds(0, _HP)], hb.at[k, pl.ds(0, _HP)], sem).wait()
        return hcarry

    lax.fori_loop(0, 2, half_body, 0)


def kernel(x, x_cord, y_cord, one_player):
    if one_player is not None:
        start = _M * jnp.asarray(one_player, dtype=jnp.int32)
        x_cord = lax.dynamic_slice_in_dim(x_cord, start, _M, axis=1)
        y_cord = lax.dynamic_slice_in_dim(y_cord, start, _M, axis=1)
    xs_all = x_cord.reshape(-1).astype(jnp.int32)
    ys_all = y_cord.reshape(-1).astype(jnp.int32)
    table = x.reshape(_B * _C * _H * _W)

    pk, mij = _consts()

    mesh = plsc.VectorSubcoreMesh(core_axis_name="c", subcore_axis_name="s")
    sc = functools.partial(
        pl.kernel,
        mesh=mesh,
        compiler_params=pltpu.CompilerParams(
            needs_layout_passes=False, use_tc_tiling_on_sc=False,
            skip_device_barrier=True),
        out_type=jax.ShapeDtypeStruct((_B * _M * _OUTP,), jnp.float32),
        scratch_types=[
            pltpu.VMEM((_M,), jnp.int32),
            pltpu.VMEM((_M,), jnp.int32),
            pltpu.VMEM((_HP,), jnp.int32),
            pltpu.VMEM((240,), jnp.int32),
            pltpu.VMEM((240,), jnp.int32),
            pltpu.VMEM((240,), jnp.float32),
            pltpu.VMEM((240,), jnp.float32),
            pltpu.VMEM((_PL + 16,), jnp.float32),
            pltpu.VMEM((2, _HB), jnp.float32),
            pltpu.SemaphoreType.DMA,
        ],
    )(_sc_body)

    out = sc(table, xs_all, ys_all, jnp.asarray(pk), jnp.asarray(mij))
    out = out.reshape(_B * _M, _OUTP)[:, :_OUT]
    return out.reshape(_B * _M, _C + 1, _N, _N)
